# baseline (speedup 1.0000x reference)
import functools

import jax
import jax.numpy as jnp
from jax import lax
from jax.experimental import pallas as pl
from jax.experimental.pallas import tpu as pltpu
from jax.experimental.pallas import tpu_sc as plsc

F = 4_000_000
NKEYS = 1_000_000
NW = 32
N1 = F // 128
L1_PER_TILE = 1024
L2_PER_TILE = 8192
L2_ROWS = (L2_PER_TILE * NW) // 8
C = 1024
NCHUNK = NKEYS // C
C_TAIL = NKEYS - NCHUNK * C

def _xb(x):
    return x ^ jnp.int32(-2147483648)

_mesh = plsc.VectorSubcoreMesh(core_axis_name="c", subcore_axis_name="s")


def _wid():
    return lax.axis_index("s") * jnp.int32(2) + lax.axis_index("c")


def _iota16():
    return lax.iota(jnp.int32, 16)


def _pack_key(a0, a1, a2):
    l0 = a0 & jnp.int32(0xFFFF)
    h0 = a0 >> jnp.int32(16)
    m = a0 * jnp.int32(21521) + a1
    t1 = m << jnp.int32(16)
    t2 = l0 * jnp.int32(65156)
    t3 = h0 * jnp.int32(-24903680)
    t4 = a1 * jnp.int32(34466) + a2
    mhi = (m >> jnp.int32(16)) & jnp.int32(0xFFFF)
    hi = jnp.int32(2) * a0 + mhi
    lo = t1
    for t in (t2, t3, t4):
        s = lo + t
        carry = (_xb(s) < _xb(t)).astype(jnp.int32)
        hi = hi + carry
        lo = s
    return hi, lo


def _le64(shi, slo_raw, khi, klob):
    slob = _xb(slo_raw)
    return (shi < khi) | ((shi == khi) & (slob <= klob))


FW = 2 * F
FROWS = F // 16
UNIT = 1024
NFULL = FROWS // UNIT
RTAIL = FROWS - NFULL * UNIT


@functools.partial(
    pl.kernel,
    mesh=_mesh,
    compiler_params=pltpu.CompilerParams(use_tc_tiling_on_sc=False, needs_layout_passes=False),
    out_type=(
        jax.ShapeDtypeStruct((L1_PER_TILE * NW * 2,), jnp.int32),
        jax.ShapeDtypeStruct((L2_ROWS, 16), jnp.int32),
        jax.ShapeDtypeStruct((FROWS, 32), jnp.int32),
    ),
    scratch_types=[
        pltpu.VMEM((16, 128), jnp.int32),
        pltpu.VMEM((2048,), jnp.int32),
        pltpu.VMEM((2048,), jnp.int32),
        pltpu.VMEM((128, 16), jnp.int32),
        pltpu.VMEM((32768,), jnp.int32),
        pltpu.VMEM((1024, 32), jnp.int32),
        pltpu.SemaphoreType.DMA,
    ],
)
def _build(ffw, l1_out, l2_out, ff_out, idxb, l1buf, s2buf, l2buf, fbuf1,
           fbuf2, sem):
    w = _wid()
    lane = _iota16()

    def fill_idx(nb, word_of_k):
        def body(t, _):
            k = t * jnp.int32(16) + lane
            idxb[lax.div(t, jnp.int32(8)),
                 pl.ds(lax.rem(t, jnp.int32(8)) * jnp.int32(16), 16)] = word_of_k(k)
            return _

        lax.fori_loop(jnp.int32(0), jnp.int32(nb * 8), body, None)

    def gather_words(nb, dst):
        cps = [
            pltpu.async_copy(ffw.at[idxb.at[jnp.int32(b)]],
                             dst.at[pl.ds(b * 128, 128)], sem)
            for b in range(nb)
        ]
        for cp in cps:
            cp.wait()

    base1 = w * jnp.int32(L1_PER_TILE)

    def l1word(k):
        s = base1 + (k >> jnp.int32(1))
        return (jnp.minimum(s * jnp.int32(256), jnp.int32(FW - 2))
                + (k & jnp.int32(1)))

    fill_idx(16, l1word)
    gather_words(16, l1buf)
    pltpu.sync_copy(l1buf, l1_out.at[pl.ds(w * jnp.int32(2 * L1_PER_TILE),
                                           2 * L1_PER_TILE)])

    for fill in range(8):
        base2 = w * jnp.int32(L2_PER_TILE) + jnp.int32(fill * 1024)

        def l2word(k, base2=base2):
            s = base2 + (k >> jnp.int32(1))
            return (jnp.minimum(s * jnp.int32(32), jnp.int32(FW - 2))
                    + (k & jnp.int32(1)))

        fill_idx(16, l2word)
        gather_words(16, s2buf)

        def torows(g, _):
            l2buf[g, pl.ds(0, 16)] = s2buf[pl.ds(g * jnp.int32(16), 16)]
            return _

        lax.fori_loop(jnp.int32(0), jnp.int32(128), torows, None)
        pltpu.sync_copy(
            l2buf,
            l2_out.at[pl.ds(w * jnp.int32(L2_PER_TILE // 8)
                            + jnp.int32(fill * 128), 128), :])

    def copy_unit(u, nrows):
        pltpu.sync_copy(ffw.at[pl.ds(u * jnp.int32(32 * UNIT), 32 * nrows)],
                        fbuf1.at[pl.ds(0, 32 * nrows)])

        def tor(g, _):
            fbuf2[lax.div(g, jnp.int32(2)),
                  pl.ds(lax.rem(g, jnp.int32(2)) * jnp.int32(16), 16)] = (
                      fbuf1[pl.ds(g * jnp.int32(16), 16)])
            return _

        lax.fori_loop(jnp.int32(0), jnp.int32(2 * nrows), tor, None)
        pltpu.sync_copy(fbuf2.at[pl.ds(0, nrows)],
                        ff_out.at[pl.ds(u * jnp.int32(UNIT), nrows), :])

    def per_tile_unit(i, _):
        u = i * jnp.int32(NW) + w

        @pl.when(u < jnp.int32(NFULL))
        def _run():
            copy_unit(u, UNIT)

        return _

    lax.fori_loop(jnp.int32(0), jnp.int32((NFULL + NW - 1) // NW),
                  per_tile_unit, None)

    @pl.when(w == jnp.int32(0))
    def _tail():
        copy_unit(jnp.int32(NFULL), RTAIL)


@functools.partial(
    pl.kernel,
    mesh=_mesh,
    compiler_params=pltpu.CompilerParams(use_tc_tiling_on_sc=False, needs_layout_passes=False),
    out_type=jax.ShapeDtypeStruct((NKEYS,), jnp.int32),
    scratch_types=[
        pltpu.VMEM((L1_PER_TILE * NW * 2,), jnp.int32),
        pltpu.VMEM((3 * C,), jnp.int32),
        pltpu.VMEM((C,), jnp.int32),
        pltpu.VMEM((C,), jnp.int32),
        pltpu.VMEM((8, 128), jnp.int32),
        pltpu.VMEM((C, 16), jnp.int32),
        pltpu.VMEM((C, 32), jnp.int32),
        pltpu.VMEM((C,), jnp.int32),
        pltpu.SemaphoreType.DMA,
    ],
)
def _search(aflat, l1f, l2rows, factrows, out, l1v, av, klov, khiv, idxb,
            l2v, fv, ov, sem):
    w = _wid()
    lane = _iota16()

    pltpu.sync_copy(l1f, l1v)

    def chunk(base, cc):
        ng = cc // 16
        nb = (cc + 127) // 128

        with jax.named_scope("dma_atoms"):
            pltpu.sync_copy(aflat.at[pl.ds(base * jnp.int32(3), 3 * cc)],
                            av.at[pl.ds(0, 3 * cc)])

        def g1(t, _):
            ib = t * jnp.int32(48) + lane * jnp.int32(3)
            a0 = plsc.load_gather(av, [ib])
            a1 = plsc.load_gather(av, [ib + 1])
            a2 = plsc.load_gather(av, [ib + 2])
            khi, klo = _pack_key(a0, a1, a2)
            klob = _xb(klo)
            pos = jnp.zeros((16,), jnp.int32)
            for step in (16384, 8192, 4096, 2048, 1024, 512, 256, 128, 64,
                         32, 16, 8, 4, 2, 1):
                np_ = pos + jnp.int32(step)
                idxc = jnp.minimum(np_, jnp.int32(N1)) - jnp.int32(1)
                slo = plsc.load_gather(l1v, [idxc * jnp.int32(2)])
                shi = plsc.load_gather(l1v, [idxc * jnp.int32(2) + jnp.int32(1)])
                ok = (np_ <= jnp.int32(N1)) & _le64(shi, slo, khi, klob)
                pos = jnp.where(ok, np_, pos)
            j1 = jnp.maximum(pos - jnp.int32(1), jnp.int32(0))
            klov[pl.ds(t * jnp.int32(16), 16)] = klo
            khiv[pl.ds(t * jnp.int32(16), 16)] = khi
            idxb[lax.div(t, jnp.int32(8)), pl.ds(lax.rem(t, jnp.int32(8)) * jnp.int32(16), 16)] = j1
            return _

        with jax.named_scope("g1_keys_l1search"):
            lax.fori_loop(jnp.int32(0), jnp.int32(ng), g1, None)

        with jax.named_scope("dma_l2"):
            copies = [
                pltpu.async_copy(l2rows.at[idxb.at[jnp.int32(r)]],
                                 l2v.at[pl.ds(r * 128, 128)], sem)
                for r in range(nb)
            ]
            for cp in copies:
                cp.wait()

        def g2(t, _):
            kk = t * jnp.int32(16) + lane
            klo = klov[pl.ds(t * jnp.int32(16), 16)]
            khi = khiv[pl.ds(t * jnp.int32(16), 16)]
            klob = _xb(klo)
            j1 = idxb[lax.div(t, jnp.int32(8)), pl.ds(lax.rem(t, jnp.int32(8)) * jnp.int32(16), 16)]
            j = jnp.zeros((16,), jnp.int32)
            for step in (4, 2, 1):
                p = j + jnp.int32(step)
                slo = plsc.load_gather(l2v, [kk, p * jnp.int32(2)])
                shi = plsc.load_gather(l2v, [kk, p * jnp.int32(2) + jnp.int32(1)])
                j = jnp.where(_le64(shi, slo, khi, klob), p, j)
            j2 = j1 * jnp.int32(8) + j
            idxb[lax.div(t, jnp.int32(8)), pl.ds(lax.rem(t, jnp.int32(8)) * jnp.int32(16), 16)] = j2
            return _

        with jax.named_scope("g2_l2search"):
            lax.fori_loop(jnp.int32(0), jnp.int32(ng), g2, None)

        with jax.named_scope("dma_fact"):
            copies = [
                pltpu.async_copy(factrows.at[idxb.at[jnp.int32(r)]],
                                 fv.at[pl.ds(r * 128, 128)], sem)
                for r in range(nb)
            ]
            for cp in copies:
                cp.wait()

        def g3(t, _):
            kk = t * jnp.int32(16) + lane
            klo = klov[pl.ds(t * jnp.int32(16), 16)]
            khi = khiv[pl.ds(t * jnp.int32(16), 16)]
            klob = _xb(klo)
            j = jnp.zeros((16,), jnp.int32)
            for step in (8, 4, 2, 1):
                p = j + jnp.int32(step)
                slo = plsc.load_gather(fv, [kk, p * jnp.int32(2)])
                shi = plsc.load_gather(fv, [kk, p * jnp.int32(2) + jnp.int32(1)])
                j = jnp.where(_le64(shi, slo, khi, klob), p, j)
            flo = plsc.load_gather(fv, [kk, j * jnp.int32(2)])
            fhi = plsc.load_gather(fv, [kk, j * jnp.int32(2) + jnp.int32(1)])
            eq = (flo == klo) & (fhi == khi)
            ov[pl.ds(t * jnp.int32(16), 16)] = eq.astype(jnp.int32)
            return _

        with jax.named_scope("g3_final"):
            lax.fori_loop(jnp.int32(0), jnp.int32(ng), g3, None)
        with jax.named_scope("dma_out"):
            pltpu.sync_copy(ov.at[pl.ds(0, cc)], out.at[pl.ds(base, cc)])

    def per_tile_chunk(i, _):
        g = i * jnp.int32(NW) + w

        @pl.when(g < jnp.int32(NCHUNK))
        def _run():
            chunk(g * jnp.int32(C), C)

        return _

    lax.fori_loop(jnp.int32(0), jnp.int32((NCHUNK + NW - 1) // NW), per_tile_chunk, None)

    @pl.when(w == jnp.int32(0))
    def _tail():
        chunk(jnp.int32(NCHUNK * C), C_TAIL)


def kernel(atoms, fact_hashes):
    a32 = atoms.astype(jnp.int32).reshape(-1)
    ffw = lax.bitcast_convert_type(fact_hashes, jnp.int32).reshape(-1)
    l1, l2, ffsc = _build(ffw)
    out = _search(a32, l1, l2, ffsc)
    return out.astype(bool)


# speedup vs baseline: 1.6857x; 1.6857x over previous
import functools

import jax
import jax.numpy as jnp
from jax import lax
from jax.experimental import pallas as pl
from jax.experimental.pallas import tpu as pltpu
from jax.experimental.pallas import tpu_sc as plsc

F = 4_000_000
NKEYS = 1_000_000
NW = 32
N1 = F // 128
L1_PER_TILE = 1024
L2_PER_TILE = 8192
L2_ROWS = (L2_PER_TILE * NW) // 8
C = 1024
NCHUNK = NKEYS // C
C_TAIL = NKEYS - NCHUNK * C

def _xb(x):
    return x ^ jnp.int32(-2147483648)

_mesh = plsc.VectorSubcoreMesh(core_axis_name="c", subcore_axis_name="s")


def _wid():
    return lax.axis_index("s") * jnp.int32(2) + lax.axis_index("c")


def _iota16():
    return lax.iota(jnp.int32, 16)


def _pack_key(a0, a1, a2):
    l0 = a0 & jnp.int32(0xFFFF)
    h0 = a0 >> jnp.int32(16)
    m = a0 * jnp.int32(21521) + a1
    t1 = m << jnp.int32(16)
    t2 = l0 * jnp.int32(65156)
    t3 = h0 * jnp.int32(-24903680)
    t4 = a1 * jnp.int32(34466) + a2
    mhi = (m >> jnp.int32(16)) & jnp.int32(0xFFFF)
    hi = jnp.int32(2) * a0 + mhi
    lo = t1
    for t in (t2, t3, t4):
        s = lo + t
        carry = (_xb(s) < _xb(t)).astype(jnp.int32)
        hi = hi + carry
        lo = s
    return hi, lo


def _le64(shi, slo_raw, khi, klob):
    slob = _xb(slo_raw)
    return (shi < khi) | ((shi == khi) & (slob <= klob))


@functools.partial(
    pl.kernel,
    mesh=_mesh,
    compiler_params=pltpu.CompilerParams(use_tc_tiling_on_sc=False, needs_layout_passes=False),
    out_type=(
        jax.ShapeDtypeStruct((L1_PER_TILE * NW * 2,), jnp.int32),
        jax.ShapeDtypeStruct((L2_ROWS, 16), jnp.int32),
    ),
    scratch_types=[
        pltpu.VMEM((8, 128), jnp.int32),
        pltpu.VMEM((1024, 32), jnp.int32),
        pltpu.VMEM((2048,), jnp.int32),
        pltpu.VMEM((128, 16), jnp.int32),
        pltpu.SemaphoreType.DMA,
    ],
)
def _build(ff, l1_out, l2_out, idxb, rows, l1buf, l2buf, sem):
    w = _wid()
    lane = _iota16()

    def gather_rows(n, row_of_s):
        def mkidx(t, _):
            s = t * jnp.int32(16) + lane
            idxb[lax.div(t, jnp.int32(8)),
                 pl.ds(lax.rem(t, jnp.int32(8)) * jnp.int32(16), 16)] = row_of_s(s)
            return _

        lax.fori_loop(jnp.int32(0), jnp.int32(n // 16), mkidx, None)
        cps = [
            pltpu.async_copy(ff.at[idxb.at[jnp.int32(r)]],
                             rows.at[pl.ds(r * 128, 128)], sem)
            for r in range(n // 128)
        ]
        for cp in cps:
            cp.wait()

    base1 = w * jnp.int32(L1_PER_TILE)

    def l1row(s):
        return jnp.minimum((base1 + s) * jnp.int32(8), jnp.int32(F // 16 - 1))

    gather_rows(L1_PER_TILE, l1row)

    def l1x(t, _):
        r = t * jnp.int32(16) + lane
        lo = plsc.load_gather(rows, [r, jnp.zeros((16,), jnp.int32)])
        hi = plsc.load_gather(rows, [r, jnp.ones((16,), jnp.int32)])
        plsc.store_scatter(l1buf, [r * jnp.int32(2)], lo)
        plsc.store_scatter(l1buf, [r * jnp.int32(2) + jnp.int32(1)], hi)
        return _

    lax.fori_loop(jnp.int32(0), jnp.int32(L1_PER_TILE // 16), l1x, None)
    pltpu.sync_copy(l1buf, l1_out.at[pl.ds(w * jnp.int32(2 * L1_PER_TILE),
                                           2 * L1_PER_TILE)])

    for rnd in range(L2_PER_TILE // 1024):
        base2 = w * jnp.int32(L2_PER_TILE) + jnp.int32(rnd * 1024)

        def l2row(s, base2=base2):
            return jnp.minimum(base2 + s, jnp.int32(F // 16 - 1))

        gather_rows(1024, l2row)

        def l2x(t, _):
            r = t * jnp.int32(16) + lane
            lo = plsc.load_gather(rows, [r, jnp.zeros((16,), jnp.int32)])
            hi = plsc.load_gather(rows, [r, jnp.ones((16,), jnp.int32)])
            orow = lax.div(r, jnp.int32(8))
            ocol = lax.rem(r, jnp.int32(8)) * jnp.int32(2)
            plsc.store_scatter(l2buf, [orow, ocol], lo)
            plsc.store_scatter(l2buf, [orow, ocol + jnp.int32(1)], hi)
            return _

        lax.fori_loop(jnp.int32(0), jnp.int32(64), l2x, None)
        pltpu.sync_copy(
            l2buf,
            l2_out.at[pl.ds(w * jnp.int32(L2_PER_TILE // 8) + jnp.int32(rnd * 128),
                            128), :])


@functools.partial(
    pl.kernel,
    mesh=_mesh,
    compiler_params=pltpu.CompilerParams(use_tc_tiling_on_sc=False, needs_layout_passes=False),
    out_type=jax.ShapeDtypeStruct((NKEYS,), jnp.int32),
    scratch_types=[
        pltpu.VMEM((L1_PER_TILE * NW * 2,), jnp.int32),
        pltpu.VMEM((3 * C,), jnp.int32),
        pltpu.VMEM((C,), jnp.int32),
        pltpu.VMEM((C,), jnp.int32),
        pltpu.VMEM((8, 128), jnp.int32),
        pltpu.VMEM((C, 16), jnp.int32),
        pltpu.VMEM((C, 32), jnp.int32),
        pltpu.VMEM((C,), jnp.int32),
        pltpu.SemaphoreType.DMA,
    ],
)
def _search(aflat, l1f, l2rows, factrows, out, l1v, av, klov, khiv, idxb,
            l2v, fv, ov, sem):
    w = _wid()
    lane = _iota16()

    pltpu.sync_copy(l1f, l1v)

    def chunk(base, cc):
        ng = cc // 16
        nb = (cc + 127) // 128

        with jax.named_scope("dma_atoms"):
            pltpu.sync_copy(aflat.at[pl.ds(base * jnp.int32(3), 3 * cc)],
                            av.at[pl.ds(0, 3 * cc)])

        def g1(t, _):
            ib = t * jnp.int32(48) + lane * jnp.int32(3)
            a0 = plsc.load_gather(av, [ib])
            a1 = plsc.load_gather(av, [ib + 1])
            a2 = plsc.load_gather(av, [ib + 2])
            khi, klo = _pack_key(a0, a1, a2)
            klob = _xb(klo)
            pos = jnp.zeros((16,), jnp.int32)
            for step in (16384, 8192, 4096, 2048, 1024, 512, 256, 128, 64,
                         32, 16, 8, 4, 2, 1):
                np_ = pos + jnp.int32(step)
                idxc = jnp.minimum(np_, jnp.int32(N1)) - jnp.int32(1)
                slo = plsc.load_gather(l1v, [idxc * jnp.int32(2)])
                shi = plsc.load_gather(l1v, [idxc * jnp.int32(2) + jnp.int32(1)])
                ok = (np_ <= jnp.int32(N1)) & _le64(shi, slo, khi, klob)
                pos = jnp.where(ok, np_, pos)
            j1 = jnp.maximum(pos - jnp.int32(1), jnp.int32(0))
            klov[pl.ds(t * jnp.int32(16), 16)] = klo
            khiv[pl.ds(t * jnp.int32(16), 16)] = khi
            idxb[lax.div(t, jnp.int32(8)), pl.ds(lax.rem(t, jnp.int32(8)) * jnp.int32(16), 16)] = j1
            return _

        with jax.named_scope("g1_keys_l1search"):
            lax.fori_loop(jnp.int32(0), jnp.int32(ng), g1, None)

        with jax.named_scope("dma_l2"):
            copies = [
                pltpu.async_copy(l2rows.at[idxb.at[jnp.int32(r)]],
                                 l2v.at[pl.ds(r * 128, 128)], sem)
                for r in range(nb)
            ]
            for cp in copies:
                cp.wait()

        def g2(t, _):
            kk = t * jnp.int32(16) + lane
            klo = klov[pl.ds(t * jnp.int32(16), 16)]
            khi = khiv[pl.ds(t * jnp.int32(16), 16)]
            klob = _xb(klo)
            j1 = idxb[lax.div(t, jnp.int32(8)), pl.ds(lax.rem(t, jnp.int32(8)) * jnp.int32(16), 16)]
            j = jnp.zeros((16,), jnp.int32)
            for step in (4, 2, 1):
                p = j + jnp.int32(step)
                slo = plsc.load_gather(l2v, [kk, p * jnp.int32(2)])
                shi = plsc.load_gather(l2v, [kk, p * jnp.int32(2) + jnp.int32(1)])
                j = jnp.where(_le64(shi, slo, khi, klob), p, j)
            j2 = j1 * jnp.int32(8) + j
            idxb[lax.div(t, jnp.int32(8)), pl.ds(lax.rem(t, jnp.int32(8)) * jnp.int32(16), 16)] = j2
            return _

        with jax.named_scope("g2_l2search"):
            lax.fori_loop(jnp.int32(0), jnp.int32(ng), g2, None)

        with jax.named_scope("dma_fact"):
            copies = [
                pltpu.async_copy(factrows.at[idxb.at[jnp.int32(r)]],
                                 fv.at[pl.ds(r * 128, 128)], sem)
                for r in range(nb)
            ]
            for cp in copies:
                cp.wait()

        def g3(t, _):
            kk = t * jnp.int32(16) + lane
            klo = klov[pl.ds(t * jnp.int32(16), 16)]
            khi = khiv[pl.ds(t * jnp.int32(16), 16)]
            klob = _xb(klo)
            j = jnp.zeros((16,), jnp.int32)
            for step in (8, 4, 2, 1):
                p = j + jnp.int32(step)
                slo = plsc.load_gather(fv, [kk, p * jnp.int32(2)])
                shi = plsc.load_gather(fv, [kk, p * jnp.int32(2) + jnp.int32(1)])
                j = jnp.where(_le64(shi, slo, khi, klob), p, j)
            flo = plsc.load_gather(fv, [kk, j * jnp.int32(2)])
            fhi = plsc.load_gather(fv, [kk, j * jnp.int32(2) + jnp.int32(1)])
            eq = (flo == klo) & (fhi == khi)
            ov[pl.ds(t * jnp.int32(16), 16)] = eq.astype(jnp.int32)
            return _

        with jax.named_scope("g3_final"):
            lax.fori_loop(jnp.int32(0), jnp.int32(ng), g3, None)
        with jax.named_scope("dma_out"):
            pltpu.sync_copy(ov.at[pl.ds(0, cc)], out.at[pl.ds(base, cc)])

    def per_tile_chunk(i, _):
        g = i * jnp.int32(NW) + w

        @pl.when(g < jnp.int32(NCHUNK))
        def _run():
            chunk(g * jnp.int32(C), C)

        return _

    lax.fori_loop(jnp.int32(0), jnp.int32((NCHUNK + NW - 1) // NW), per_tile_chunk, None)

    @pl.when(w == jnp.int32(0))
    def _tail():
        chunk(jnp.int32(NCHUNK * C), C_TAIL)


def kernel(atoms, fact_hashes):
    a32 = atoms.astype(jnp.int32).reshape(-1)
    ff = lax.bitcast_convert_type(fact_hashes, jnp.int32).reshape(F // 16, 32)
    l1, l2 = _build(ff)
    out = _search(a32, l1, l2, ff)
    return out.astype(bool)


# speedup vs baseline: 1.7825x; 1.0574x over previous
import functools

import jax
import jax.numpy as jnp
from jax import lax
from jax.experimental import pallas as pl
from jax.experimental.pallas import tpu as pltpu
from jax.experimental.pallas import tpu_sc as plsc

F = 4_000_000
NKEYS = 1_000_000
NW = 32
N1 = F // 128
L1_PER_TILE = 1024
L2_PER_TILE = 8192
L2_ROWS = (L2_PER_TILE * NW) // 8
C = 1024
NCHUNK = NKEYS // C
C_TAIL = NKEYS - NCHUNK * C

def _xb(x):
    return x ^ jnp.int32(-2147483648)

_mesh = plsc.VectorSubcoreMesh(core_axis_name="c", subcore_axis_name="s")


def _wid():
    return lax.axis_index("s") * jnp.int32(2) + lax.axis_index("c")


def _iota16():
    return lax.iota(jnp.int32, 16)


def _pack_key(a0, a1, a2):
    l0 = a0 & jnp.int32(0xFFFF)
    h0 = a0 >> jnp.int32(16)
    m = a0 * jnp.int32(21521) + a1
    t1 = m << jnp.int32(16)
    t2 = l0 * jnp.int32(65156)
    t3 = h0 * jnp.int32(-24903680)
    t4 = a1 * jnp.int32(34466) + a2
    mhi = (m >> jnp.int32(16)) & jnp.int32(0xFFFF)
    hi = jnp.int32(2) * a0 + mhi
    lo = t1
    for t in (t2, t3, t4):
        s = lo + t
        carry = (_xb(s) < _xb(t)).astype(jnp.int32)
        hi = hi + carry
        lo = s
    return hi, lo


def _le64(shi, slo_raw, khi, klob):
    slob = _xb(slo_raw)
    return (shi < khi) | ((shi == khi) & (slob <= klob))


@functools.partial(
    pl.kernel,
    mesh=_mesh,
    compiler_params=pltpu.CompilerParams(use_tc_tiling_on_sc=False, needs_layout_passes=False),
    out_type=(
        jax.ShapeDtypeStruct((L1_PER_TILE * NW * 2,), jnp.int32),
        jax.ShapeDtypeStruct((L2_ROWS, 16), jnp.int32),
    ),
    scratch_types=[
        pltpu.VMEM((8, 128), jnp.int32),
        pltpu.VMEM((1024, 32), jnp.int32),
        pltpu.VMEM((2048,), jnp.int32),
        pltpu.VMEM((128, 16), jnp.int32),
        pltpu.SemaphoreType.DMA,
    ],
)
def _build(ff, l1_out, l2_out, idxb, rows, l1buf, l2buf, sem):
    w = _wid()
    lane = _iota16()

    def gather_rows(n, row_of_s):
        def mkidx(t, _):
            s = t * jnp.int32(16) + lane
            idxb[lax.div(t, jnp.int32(8)),
                 pl.ds(lax.rem(t, jnp.int32(8)) * jnp.int32(16), 16)] = row_of_s(s)
            return _

        lax.fori_loop(jnp.int32(0), jnp.int32(n // 16), mkidx, None)
        cps = [
            pltpu.async_copy(ff.at[idxb.at[jnp.int32(r)]],
                             rows.at[pl.ds(r * 128, 128)], sem)
            for r in range(n // 128)
        ]
        for cp in cps:
            cp.wait()

    base1 = w * jnp.int32(L1_PER_TILE)

    def l1row(s):
        return jnp.minimum((base1 + s) * jnp.int32(8), jnp.int32(F // 16 - 1))

    gather_rows(L1_PER_TILE, l1row)

    def l1x(t, _):
        r = t * jnp.int32(16) + lane
        lo = plsc.load_gather(rows, [r, jnp.zeros((16,), jnp.int32)])
        hi = plsc.load_gather(rows, [r, jnp.ones((16,), jnp.int32)])
        plsc.store_scatter(l1buf, [r * jnp.int32(2)], lo)
        plsc.store_scatter(l1buf, [r * jnp.int32(2) + jnp.int32(1)], hi)
        return _

    lax.fori_loop(jnp.int32(0), jnp.int32(L1_PER_TILE // 16), l1x, None)
    pltpu.sync_copy(l1buf, l1_out.at[pl.ds(w * jnp.int32(2 * L1_PER_TILE),
                                           2 * L1_PER_TILE)])

    for rnd in range(L2_PER_TILE // 1024):
        base2 = w * jnp.int32(L2_PER_TILE) + jnp.int32(rnd * 1024)

        def l2row(s, base2=base2):
            return jnp.minimum(base2 + s, jnp.int32(F // 16 - 1))

        gather_rows(1024, l2row)

        def l2x(t, _):
            r = t * jnp.int32(16) + lane
            lo = plsc.load_gather(rows, [r, jnp.zeros((16,), jnp.int32)])
            hi = plsc.load_gather(rows, [r, jnp.ones((16,), jnp.int32)])
            orow = lax.div(r, jnp.int32(8))
            ocol = lax.rem(r, jnp.int32(8)) * jnp.int32(2)
            plsc.store_scatter(l2buf, [orow, ocol], lo)
            plsc.store_scatter(l2buf, [orow, ocol + jnp.int32(1)], hi)
            return _

        lax.fori_loop(jnp.int32(0), jnp.int32(64), l2x, None)
        pltpu.sync_copy(
            l2buf,
            l2_out.at[pl.ds(w * jnp.int32(L2_PER_TILE // 8) + jnp.int32(rnd * 128),
                            128), :])


@functools.partial(
    pl.kernel,
    mesh=_mesh,
    compiler_params=pltpu.CompilerParams(use_tc_tiling_on_sc=False, needs_layout_passes=False),
    out_type=jax.ShapeDtypeStruct((NKEYS,), jnp.int32),
    scratch_types=[
        pltpu.VMEM((L1_PER_TILE * NW * 2,), jnp.int32),
        pltpu.VMEM((3 * C,), jnp.int32),
        pltpu.VMEM((C,), jnp.int32),
        pltpu.VMEM((C,), jnp.int32),
        pltpu.VMEM((8, 128), jnp.int32),
        pltpu.VMEM((C, 16), jnp.int32),
        pltpu.VMEM((C, 32), jnp.int32),
        pltpu.VMEM((C,), jnp.int32),
        pltpu.VMEM((4112,), jnp.int32),
        pltpu.SemaphoreType.DMA,
    ],
)
def _search(aflat, l1f, l2rows, factrows, out, l1v, av, klov, khiv, idxb,
            l2v, fv, ov, bkt, sem):
    w = _wid()
    lane = _iota16()

    pltpu.sync_copy(l1f, l1v)

    def mkbkt(t, _):
        b = t * jnp.int32(16) + lane
        bhi = jnp.minimum(b, jnp.int32(4096)) * jnp.int32(64)
        pos = jnp.zeros((16,), jnp.int32)
        for step in (16384, 8192, 4096, 2048, 1024, 512, 256, 128, 64,
                     32, 16, 8, 4, 2, 1):
            np_ = pos + jnp.int32(step)
            idxc = jnp.minimum(np_, jnp.int32(N1)) - jnp.int32(1)
            shi = plsc.load_gather(l1v, [idxc * jnp.int32(2) + jnp.int32(1)])
            ok = (np_ <= jnp.int32(N1)) & (shi < bhi)
            pos = jnp.where(ok, np_, pos)
        bkt[pl.ds(t * jnp.int32(16), 16)] = pos
        return _

    lax.fori_loop(jnp.int32(0), jnp.int32(257), mkbkt, None)

    def chunk(base, cc):
        ng = cc // 16
        nb = (cc + 127) // 128

        with jax.named_scope("dma_atoms"):
            pltpu.sync_copy(aflat.at[pl.ds(base * jnp.int32(3), 3 * cc)],
                            av.at[pl.ds(0, 3 * cc)])

        def g1(t, _):
            ib = t * jnp.int32(48) + lane * jnp.int32(3)
            a0 = plsc.load_gather(av, [ib])
            a1 = plsc.load_gather(av, [ib + 1])
            a2 = plsc.load_gather(av, [ib + 2])
            khi, klo = _pack_key(a0, a1, a2)
            klob = _xb(klo)
            b = khi >> jnp.int32(6)
            lo = plsc.load_gather(bkt, [b]) - jnp.int32(1)
            hi = plsc.load_gather(bkt, [b + jnp.int32(1)])

            def srch_cond(c):
                lo_, hi_ = c
                return jnp.max(hi_ - lo_) > jnp.int32(1)

            def srch_body(c):
                lo_, hi_ = c
                upd = (hi_ - lo_) > jnp.int32(1)
                mid = (lo_ + hi_) >> jnp.int32(1)
                midc = jnp.maximum(mid, jnp.int32(0))
                slo = plsc.load_gather(l1v, [midc * jnp.int32(2)])
                shi = plsc.load_gather(l1v, [midc * jnp.int32(2) + jnp.int32(1)])
                le = _le64(shi, slo, khi, klob)
                lo_ = jnp.where(upd & le, mid, lo_)
                hi_ = jnp.where(upd & (~le), mid, hi_)
                return lo_, hi_

            lo, hi = lax.while_loop(srch_cond, srch_body, (lo, hi))
            j1 = jnp.maximum(lo, jnp.int32(0))
            klov[pl.ds(t * jnp.int32(16), 16)] = klo
            khiv[pl.ds(t * jnp.int32(16), 16)] = khi
            idxb[lax.div(t, jnp.int32(8)), pl.ds(lax.rem(t, jnp.int32(8)) * jnp.int32(16), 16)] = j1
            return _

        with jax.named_scope("g1_keys_l1search"):
            lax.fori_loop(jnp.int32(0), jnp.int32(ng), g1, None)

        with jax.named_scope("dma_l2"):
            copies = [
                pltpu.async_copy(l2rows.at[idxb.at[jnp.int32(r)]],
                                 l2v.at[pl.ds(r * 128, 128)], sem)
                for r in range(nb)
            ]
            for cp in copies:
                cp.wait()

        def g2(t, _):
            kk = t * jnp.int32(16) + lane
            klo = klov[pl.ds(t * jnp.int32(16), 16)]
            khi = khiv[pl.ds(t * jnp.int32(16), 16)]
            klob = _xb(klo)
            j1 = idxb[lax.div(t, jnp.int32(8)), pl.ds(lax.rem(t, jnp.int32(8)) * jnp.int32(16), 16)]
            j = jnp.zeros((16,), jnp.int32)
            for step in (4, 2, 1):
                p = j + jnp.int32(step)
                slo = plsc.load_gather(l2v, [kk, p * jnp.int32(2)])
                shi = plsc.load_gather(l2v, [kk, p * jnp.int32(2) + jnp.int32(1)])
                j = jnp.where(_le64(shi, slo, khi, klob), p, j)
            j2 = j1 * jnp.int32(8) + j
            idxb[lax.div(t, jnp.int32(8)), pl.ds(lax.rem(t, jnp.int32(8)) * jnp.int32(16), 16)] = j2
            return _

        with jax.named_scope("g2_l2search"):
            lax.fori_loop(jnp.int32(0), jnp.int32(ng), g2, None)

        with jax.named_scope("dma_fact"):
            copies = [
                pltpu.async_copy(factrows.at[idxb.at[jnp.int32(r)]],
                                 fv.at[pl.ds(r * 128, 128)], sem)
                for r in range(nb)
            ]
            for cp in copies:
                cp.wait()

        def g3(t, _):
            kk = t * jnp.int32(16) + lane
            klo = klov[pl.ds(t * jnp.int32(16), 16)]
            khi = khiv[pl.ds(t * jnp.int32(16), 16)]
            klob = _xb(klo)
            j = jnp.zeros((16,), jnp.int32)
            for step in (8, 4, 2, 1):
                p = j + jnp.int32(step)
                slo = plsc.load_gather(fv, [kk, p * jnp.int32(2)])
                shi = plsc.load_gather(fv, [kk, p * jnp.int32(2) + jnp.int32(1)])
                j = jnp.where(_le64(shi, slo, khi, klob), p, j)
            flo = plsc.load_gather(fv, [kk, j * jnp.int32(2)])
            fhi = plsc.load_gather(fv, [kk, j * jnp.int32(2) + jnp.int32(1)])
            eq = (flo == klo) & (fhi == khi)
            ov[pl.ds(t * jnp.int32(16), 16)] = eq.astype(jnp.int32)
            return _

        with jax.named_scope("g3_final"):
            lax.fori_loop(jnp.int32(0), jnp.int32(ng), g3, None)
        with jax.named_scope("dma_out"):
            pltpu.sync_copy(ov.at[pl.ds(0, cc)], out.at[pl.ds(base, cc)])

    def per_tile_chunk(i, _):
        g = i * jnp.int32(NW) + w

        @pl.when(g < jnp.int32(NCHUNK))
        def _run():
            chunk(g * jnp.int32(C), C)

        return _

    lax.fori_loop(jnp.int32(0), jnp.int32((NCHUNK + NW - 1) // NW), per_tile_chunk, None)

    @pl.when(w == jnp.int32(0))
    def _tail():
        chunk(jnp.int32(NCHUNK * C), C_TAIL)


def kernel(atoms, fact_hashes):
    a32 = atoms.astype(jnp.int32).reshape(-1)
    ff = lax.bitcast_convert_type(fact_hashes, jnp.int32).reshape(F // 16, 32)
    l1, l2 = _build(ff)
    out = _search(a32, l1, l2, ff)
    return out.astype(bool)
